# baseline (speedup 1.0000x reference)
import jax
import jax.numpy as jnp
from jax import lax
from jax.experimental import pallas as pl
from jax.experimental.pallas import tpu as pltpu
from jax.experimental.pallas import tpu_sc as plsc

VOCAB = 1000000
EMBED = 64
BATCH = 16384

_info = plsc.get_sparse_core_info()
NC, NS, L = _info.num_cores, _info.num_subcores, _info.num_lanes
NW = NC * NS
BPW = BATCH // NW
CHUNK = 32
NCHUNK = BPW // CHUNK
NVREG = EMBED // 16


def _sc_body(x_hbm, t_hbm, tgt_hbm, ctx_hbm, out_hbm,
             xidx, tidx, tbufs, cbufs, outv, semts, semcs):
    wid = lax.axis_index("s") * NC + lax.axis_index("c")
    base = wid * BPW

    pltpu.sync_copy(x_hbm.at[pl.ds(base, BPW)], xidx)
    pltpu.sync_copy(t_hbm.at[pl.ds(base, BPW)], tidx)

    lane = lax.iota(jnp.int32, L)

    def fire(p, slot):
        cbase = p * CHUNK
        for cc in range(CHUNK // 16):
            tv = tidx[pl.ds(cbase + cc * 16, 16)]
            xv = xidx[pl.ds(cbase + cc * 16, 16)]
            for j in range(16):
                row = cc * 16 + j
                pltpu.async_copy(
                    tgt_hbm.at[tv[j]],
                    tbufs[slot].at[row, pl.ds(0, EMBED)], semts[slot])
                pltpu.async_copy(
                    ctx_hbm.at[xv[j]],
                    cbufs[slot].at[row, pl.ds(0, EMBED)], semcs[slot])

    def drain(slot):
        for row in range(CHUNK):
            pltpu.make_async_copy(
                tgt_hbm.at[0], tbufs[slot].at[row, pl.ds(0, EMBED)],
                semts[slot]).wait()
            pltpu.make_async_copy(
                ctx_hbm.at[0], cbufs[slot].at[row, pl.ds(0, EMBED)],
                semcs[slot]).wait()

    def compute(p, slot):
        cbase = p * CHUNK
        tbuf, cbuf = tbufs[slot], cbufs[slot]
        for cc in range(CHUNK // 16):
            res = jnp.zeros((L,), jnp.float32)
            for j in range(16):
                row = cc * 16 + j
                s = jnp.zeros((L,), jnp.float32)
                for k in range(NVREG):
                    s = s + (tbuf[row, pl.ds(k * 16, 16)]
                             * cbuf[row, pl.ds(k * 16, 16)])
                tot = jnp.sum(s)
                res = jnp.where(lane == j, tot, res)
            outv[pl.ds(cbase + cc * 16, 16)] = res

    fire(0, 0)

    def pair_step(q, carry):
        fire(2 * q + 1, 1)
        drain(0)
        compute(2 * q, 0)

        @pl.when(2 * q + 2 < NCHUNK)
        def _():
            fire(2 * q + 2, 0)
        drain(1)
        compute(2 * q + 1, 1)
        return carry

    lax.fori_loop(0, NCHUNK // 2, pair_step, 0)
    pltpu.sync_copy(outv, out_hbm.at[pl.ds(base, BPW)])


@jax.jit
def _sc_call(x, t, target_table, context_table):
    mesh = plsc.VectorSubcoreMesh(core_axis_name="c", subcore_axis_name="s")
    return pl.kernel(
        _sc_body,
        out_type=jax.ShapeDtypeStruct((BATCH,), jnp.float32),
        mesh=mesh,
        compiler_params=pltpu.CompilerParams(
            needs_layout_passes=False,
        ),
        scratch_types=[
            pltpu.VMEM((BPW,), jnp.int32),
            pltpu.VMEM((BPW,), jnp.int32),
            [pltpu.VMEM((CHUNK, 2 * EMBED), jnp.float32) for _ in range(2)],
            [pltpu.VMEM((CHUNK, 2 * EMBED), jnp.float32) for _ in range(2)],
            pltpu.VMEM((BPW,), jnp.float32),
            [pltpu.SemaphoreType.DMA for _ in range(2)],
            [pltpu.SemaphoreType.DMA for _ in range(2)],
        ],
    )(x, t, target_table, context_table)


def kernel(x, t, target_table, context_table):
    tt, cc = jax.lax.optimization_barrier(
        (target_table.T, context_table.T))
    return _sc_call(x, t, tt.T, cc.T)


# speedup vs baseline: 1.0180x; 1.0180x over previous
import jax
import jax.numpy as jnp
from jax import lax
from jax.experimental import pallas as pl
from jax.experimental.pallas import tpu as pltpu
from jax.experimental.pallas import tpu_sc as plsc

VOCAB = 1000000
EMBED = 64
BATCH = 16384

_info = plsc.get_sparse_core_info()
NC, NS, L = _info.num_cores, _info.num_subcores, _info.num_lanes
NW = NC * NS
BPW = BATCH // NW
CHUNK = 32
NCHUNK = BPW // CHUNK
NVREG = EMBED // 16


def _sc_body(x_hbm, t_hbm, tgt_hbm, ctx_hbm, out_hbm,
             xidx, tidx, tbuf, cbuf, outv, semt, semc):
    wid = lax.axis_index("s") * NC + lax.axis_index("c")
    base = wid * BPW

    pltpu.sync_copy(x_hbm.at[pl.ds(base, BPW)], xidx)
    pltpu.sync_copy(t_hbm.at[pl.ds(base, BPW)], tidx)

    lane = lax.iota(jnp.int32, L)

    def chunk_step(p, carry):
        cbase = p * CHUNK
        descs = []
        for cc in range(CHUNK // 16):
            tv = tidx[pl.ds(cbase + cc * 16, 16)]
            xv = xidx[pl.ds(cbase + cc * 16, 16)]
            for j in range(16):
                row = cc * 16 + j
                descs.append(pltpu.async_copy(
                    tgt_hbm.at[tv[j]], tbuf.at[row, pl.ds(0, EMBED)], semt))
                descs.append(pltpu.async_copy(
                    ctx_hbm.at[xv[j]], cbuf.at[row, pl.ds(0, EMBED)], semc))
        for d in descs:
            d.wait()
        for cc in range(CHUNK // 16):
            res = jnp.zeros((L,), jnp.float32)
            for j in range(16):
                row = cc * 16 + j
                s = jnp.zeros((L,), jnp.float32)
                for k in range(NVREG):
                    s = s + (tbuf[row, pl.ds(k * 16, 16)]
                             * cbuf[row, pl.ds(k * 16, 16)])
                tot = jnp.sum(s)
                res = jnp.where(lane == j, tot, res)
            outv[pl.ds(cbase + cc * 16, 16)] = res
        return carry

    lax.fori_loop(0, NCHUNK, chunk_step, 0)
    pltpu.sync_copy(outv, out_hbm.at[pl.ds(base, BPW)])


@jax.jit
def _sc_call(x, t, target_table, context_table):
    mesh = plsc.VectorSubcoreMesh(core_axis_name="c", subcore_axis_name="s")
    return pl.kernel(
        _sc_body,
        out_type=jax.ShapeDtypeStruct((BATCH,), jnp.float32),
        mesh=mesh,
        compiler_params=pltpu.CompilerParams(
            needs_layout_passes=False,
        ),
        scratch_types=[
            pltpu.VMEM((BPW,), jnp.int32),
            pltpu.VMEM((BPW,), jnp.int32),
            pltpu.VMEM((CHUNK, 2 * EMBED), jnp.float32),
            pltpu.VMEM((CHUNK, 2 * EMBED), jnp.float32),
            pltpu.VMEM((BPW,), jnp.float32),
            pltpu.SemaphoreType.DMA,
            pltpu.SemaphoreType.DMA,
        ],
    )(x, t, target_table, context_table)


def kernel(x, t, target_table, context_table):
    tt, cc = jax.lax.optimization_barrier(
        (target_table.T, context_table.T))
    return _sc_call(x, t, tt.T, cc.T)
